# baseline (speedup 1.0000x reference)
import jax
import jax.numpy as jnp
from jax import lax
from jax.experimental import pallas as pl
from jax.experimental.pallas import tpu as pltpu
from jax.experimental.pallas import tpu_sc as plsc

BATCH = 4096
SEQ = 200
EMBED = 64
TOTAL = BATCH * SEQ
NC = 2
NS = 16
NW = NC * NS
PER_W = TOTAL // NW
CHUNK = 128
NCHUNK = PER_W // CHUNK


def _gather_body(idx_hbm, table_hbm, out_hbm, idx_v, rows_v, gsem):
    wid = lax.axis_index("s") * NC + lax.axis_index("c")
    base = wid * PER_W
    pltpu.sync_copy(idx_hbm.at[wid], idx_v)

    def body(j, carry):
        pltpu.async_copy(table_hbm.at[idx_v.at[j]], rows_v, gsem).wait()
        pltpu.sync_copy(rows_v, out_hbm.at[pl.ds(base + j * CHUNK, CHUNK)])
        return carry

    lax.fori_loop(0, NCHUNK, body, 0)


def kernel(input_ids, table):
    idx = input_ids.reshape(NW, NCHUNK, CHUNK)
    mesh = plsc.VectorSubcoreMesh(core_axis_name="c", subcore_axis_name="s")
    out = pl.kernel(
        _gather_body,
        out_type=jax.ShapeDtypeStruct((TOTAL, EMBED), jnp.float32),
        mesh=mesh,
        scratch_types=[
            pltpu.VMEM((NCHUNK, CHUNK), jnp.int32),
            pltpu.VMEM((CHUNK, EMBED), jnp.float32),
            pltpu.SemaphoreType.DMA,
        ],
        compiler_params=pltpu.CompilerParams(use_tc_tiling_on_sc=False),
    )(idx, table)
    return out.reshape(BATCH, SEQ, EMBED)


# speedup vs baseline: 1.0894x; 1.0894x over previous
import jax
import jax.numpy as jnp
from jax import lax
from jax.experimental import pallas as pl
from jax.experimental.pallas import tpu as pltpu
from jax.experimental.pallas import tpu_sc as plsc

BATCH = 4096
SEQ = 200
EMBED = 64
TOTAL = BATCH * SEQ
NC = 2
NS = 16
NW = NC * NS
PER_W = TOTAL // NW
CHUNK = 128
NCHUNK = PER_W // CHUNK


H = 4
NBUF = 2 * H
NGROUP = NCHUNK // H
NPAIR = NGROUP // 2


def _gather_body(idx_hbm, table_hbm, out_hbm, idx_v, rows, *sems):
    gsem = sems[:NBUF]
    wsem = sems[NBUF:]
    wid = lax.axis_index("s") * NC + lax.axis_index("c")
    base = wid * PER_W
    pltpu.sync_copy(idx_hbm.at[wid], idx_v)

    def gather_start(j, b):
        pltpu.async_copy(table_hbm.at[idx_v.at[j]], rows.at[b], gsem[b])

    def gather_wait(j, b):
        pltpu.make_async_copy(table_hbm.at[idx_v.at[j]], rows.at[b],
                              gsem[b]).wait()

    def write_start(j, b):
        pltpu.async_copy(rows.at[b],
                         out_hbm.at[pl.ds(base + j * CHUNK, CHUNK)], wsem[b])

    def write_wait(j, b):
        pltpu.make_async_copy(rows.at[b],
                              out_hbm.at[pl.ds(base + j * CHUNK, CHUNK)],
                              wsem[b]).wait()

    for b in range(H):
        gather_start(b, b)

    def body(u, carry):
        for p in (0, 1):
            g = 2 * u + p
            c0 = g * H
            for b in range(H):
                gather_wait(c0 + b, p * H + b)
            for b in range(H):
                write_start(c0 + b, p * H + b)

            @pl.when(g >= 1)
            def _(c0=c0, p=p):
                for b in range(H):
                    write_wait(c0 - H + b, (1 - p) * H + b)

            @pl.when(g < NGROUP - 1)
            def _(c0=c0, p=p):
                for b in range(H):
                    gather_start(c0 + H + b, (1 - p) * H + b)
        return carry

    lax.fori_loop(0, NPAIR, body, 0)

    for b in range(H):
        write_wait(NCHUNK - H + b, H + b)


def kernel(input_ids, table):
    idx = input_ids.reshape(NW, NCHUNK, CHUNK)
    mesh = plsc.VectorSubcoreMesh(core_axis_name="c", subcore_axis_name="s")
    out = pl.kernel(
        _gather_body,
        out_type=jax.ShapeDtypeStruct((TOTAL, EMBED), jnp.float32),
        mesh=mesh,
        scratch_types=(
            [pltpu.VMEM((NCHUNK, CHUNK), jnp.int32),
             pltpu.VMEM((NBUF, CHUNK, EMBED), jnp.float32)]
            + [pltpu.SemaphoreType.DMA] * (2 * NBUF)
        ),
        compiler_params=pltpu.CompilerParams(use_tc_tiling_on_sc=False),
    )(idx, table)
    return out.reshape(BATCH, SEQ, EMBED)
